# baseline (speedup 1.0000x reference)
import functools

import jax
import jax.numpy as jnp
from jax import lax
from jax.experimental import pallas as pl
from jax.experimental.pallas import tpu as pltpu
from jax.experimental.pallas import tpu_sc as plsc

_N_FRAMES = 8


def kernel(x, number):
    del number
    B, F, D = x.shape
    n = _N_FRAMES
    step = F // (n + 1)

    info = plsc.get_sparse_core_info()
    NC, NS = info.num_cores, info.num_subcores
    assert B == NC * NS

    mesh = plsc.VectorSubcoreMesh(core_axis_name="c", subcore_axis_name="s")

    @functools.partial(
        pl.kernel,
        mesh=mesh,
        out_type=jax.ShapeDtypeStruct((B, n, D), x.dtype),
        scratch_types=[pltpu.SemaphoreType.DMA],
    )
    def copy_frames(x_hbm, out_hbm, sem):
        wid = lax.axis_index("s") * NC + lax.axis_index("c")
        copies = [
            pltpu.make_async_copy(
                x_hbm.at[wid, (j + 1) * step], out_hbm.at[wid, j], sem
            )
            for j in range(n)
        ]
        for c in copies:
            c.start()
        for c in copies:
            c.wait()

    return copy_frames(x)


# speedup vs baseline: 2.5981x; 2.5981x over previous
import functools

import jax
import jax.numpy as jnp
from jax import lax
from jax.experimental import pallas as pl
from jax.experimental.pallas import tpu as pltpu
from jax.experimental.pallas import tpu_sc as plsc

_N_FRAMES = 8


def kernel(x, number):
    del number
    B, F, D = x.shape
    n = _N_FRAMES
    step = F // (n + 1)

    info = plsc.get_sparse_core_info()
    NS = info.num_subcores
    rows_total = B * n
    rpw = rows_total // NS
    assert rpw == 16 and (rpw * NS) == rows_total

    x2d = x.reshape(B * F, D)

    mesh = plsc.VectorSubcoreMesh(
        core_axis_name="c", subcore_axis_name="s", num_cores=1
    )

    @functools.partial(
        pl.kernel,
        mesh=mesh,
        out_type=jax.ShapeDtypeStruct((rows_total, D), x.dtype),
        scratch_types=[
            pltpu.VMEM((rpw, D), x.dtype),
            pltpu.SemaphoreType.DMA,
        ],
    )
    def gather_rows(table_hbm, out_hbm, rows_v, sem):
        w = lax.axis_index("s")
        base = w * rpw
        r = base + lax.iota(jnp.int32, 16)
        b = r >> 3
        j = (r & 7) + 1
        idx = b * F + j * step
        pltpu.async_copy(table_hbm.at[idx], rows_v, sem).wait()
        pltpu.sync_copy(rows_v, out_hbm.at[pl.ds(base, rpw)])

    return gather_rows(x2d).reshape(B, n, D)
